# baseline (speedup 1.0000x reference)
import functools

import jax
import jax.numpy as jnp
from jax import lax
from jax.experimental import pallas as pl
from jax.experimental.pallas import tpu as pltpu
from jax.experimental.pallas import tpu_sc as plsc


def kernel(x):
    B, S, C = x.shape
    h = int(round(S ** 0.5))
    x4 = x.reshape(B, h, h, C)

    info = plsc.get_sparse_core_info()
    nw = info.num_cores * info.num_subcores
    per_w = B // nw

    mesh = plsc.VectorSubcoreMesh(core_axis_name="c", subcore_axis_name="s")

    @functools.partial(
        pl.kernel,
        mesh=mesh,
        out_type=jax.ShapeDtypeStruct((B, h - 1, h - 2, C), jnp.float32),
        compiler_params=pltpu.CompilerParams(use_tc_tiling_on_sc=False),
    )
    def copy_k(x_hbm, out_hbm):
        wid = lax.axis_index("s") * info.num_cores + lax.axis_index("c")
        for t in range(per_w):
            b = wid * per_w + t
            pltpu.sync_copy(
                x_hbm.at[b, pl.ds(0, h - 1), pl.ds(1, h - 2), :],
                out_hbm.at[b],
            )

    out4 = copy_k(x4)
    return out4.reshape(B, (h - 1) * (h - 2), C)


# speedup vs baseline: 3.5717x; 3.5717x over previous
import functools

import jax
import jax.numpy as jnp
from jax import lax
from jax.experimental import pallas as pl
from jax.experimental.pallas import tpu as pltpu
from jax.experimental.pallas import tpu_sc as plsc


def kernel(x):
    B, S, C = x.shape
    h = int(round(S ** 0.5))
    hi, hj = h - 1, h - 2
    x4 = x.reshape(B, h, h, C)

    info = plsc.get_sparse_core_info()
    nw = info.num_cores * info.num_subcores
    per_w = B // nw

    CH = 8
    chunks = []
    i0 = 0
    while i0 < hi:
        chunks.append((i0, min(CH, hi - i0)))
        i0 += CH
    items = [(t, c0, cnt) for t in range(per_w) for (c0, cnt) in chunks]
    n_items = len(items)

    mesh = plsc.VectorSubcoreMesh(core_axis_name="c", subcore_axis_name="s")

    @functools.partial(
        pl.kernel,
        mesh=mesh,
        out_type=jax.ShapeDtypeStruct((B, hi, hj, C), jnp.float32),
        scratch_types=[
            pltpu.VMEM((CH, hj, C), jnp.float32),
            pltpu.VMEM((CH, hj, C), jnp.float32),
            pltpu.SemaphoreType.DMA,
            pltpu.SemaphoreType.DMA,
            pltpu.SemaphoreType.DMA,
            pltpu.SemaphoreType.DMA,
        ],
        compiler_params=pltpu.CompilerParams(use_tc_tiling_on_sc=False),
    )
    def copy_k(x_hbm, out_hbm, buf0, buf1, in0, in1, o0, o1):
        wid = lax.axis_index("s") * info.num_cores + lax.axis_index("c")
        bufs = (buf0, buf1)
        in_sems = (in0, in1)
        out_sems = (o0, o1)

        def in_copy(k):
            t, c0, cnt = items[k]
            b = wid * per_w + t
            return pltpu.make_async_copy(
                x_hbm.at[b, pl.ds(c0, cnt), pl.ds(1, hj), :],
                bufs[k % 2].at[pl.ds(0, cnt)],
                in_sems[k % 2],
            )

        def out_copy(k):
            t, c0, cnt = items[k]
            b = wid * per_w + t
            return pltpu.make_async_copy(
                bufs[k % 2].at[pl.ds(0, cnt)],
                out_hbm.at[b, pl.ds(c0, cnt)],
                out_sems[k % 2],
            )

        in_copy(0).start()
        if n_items > 1:
            in_copy(1).start()
        for k in range(n_items):
            in_copy(k).wait()
            out_copy(k).start()
            if k + 2 < n_items:
                out_copy(k).wait()
                in_copy(k + 2).start()
        if n_items >= 2:
            out_copy(n_items - 2).wait()
        out_copy(n_items - 1).wait()

    out4 = copy_k(x4)
    return out4.reshape(B, hi * hj, C)


# speedup vs baseline: 4.2354x; 1.1858x over previous
import functools

import jax
import jax.numpy as jnp
from jax import lax
from jax.experimental import pallas as pl
from jax.experimental.pallas import tpu as pltpu
from jax.experimental.pallas import tpu_sc as plsc


def kernel(x):
    B, S, C = x.shape
    h = int(round(S ** 0.5))
    hi, hj = h - 1, h - 2
    K = hi * hj

    info = plsc.get_sparse_core_info()
    nw = info.num_cores * info.num_subcores
    per_w = B // nw

    GR = 8
    chunks = []
    i0 = 0
    while i0 < hi:
        chunks.append((i0, min(GR, hi - i0)))
        i0 += GR
    items = [(t, c0, ngr) for t in range(per_w) for (c0, ngr) in chunks]
    n_items = len(items)

    mesh = plsc.VectorSubcoreMesh(core_axis_name="c", subcore_axis_name="s")

    @functools.partial(
        pl.kernel,
        mesh=mesh,
        out_type=jax.ShapeDtypeStruct((B, K, C), jnp.float32),
        scratch_types=[
            pltpu.VMEM((GR * hj, C), jnp.float32),
            pltpu.VMEM((GR * hj, C), jnp.float32),
            pltpu.SemaphoreType.DMA,
            pltpu.SemaphoreType.DMA,
            pltpu.SemaphoreType.DMA,
            pltpu.SemaphoreType.DMA,
        ],
        compiler_params=pltpu.CompilerParams(use_tc_tiling_on_sc=False),
    )
    def copy_k(x_hbm, out_hbm, ob0, ob1, is0, is1, os0, os1):
        wid = lax.axis_index("s") * info.num_cores + lax.axis_index("c")
        obufs = (ob0, ob1)
        isems, osems = (is0, is1), (os0, os1)

        def gather_chunk(k):
            t, c0, ngr = items[k]
            b = wid * per_w + t
            cps = [
                pltpu.make_async_copy(
                    x_hbm.at[b, pl.ds((c0 + q) * h + 1, hj), :],
                    obufs[k % 2].at[pl.ds(q * hj, hj)],
                    isems[k % 2],
                )
                for q in range(ngr)
            ]
            for cp in cps:
                cp.start()
            return cps

        def wait_chunk(cps):
            for cp in cps:
                cp.wait()

        def out_copy(k):
            t, c0, ngr = items[k]
            b = wid * per_w + t
            return pltpu.make_async_copy(
                obufs[k % 2].at[pl.ds(0, ngr * hj)],
                out_hbm.at[b, pl.ds(c0 * hj, ngr * hj), :],
                osems[k % 2],
            )

        pend = gather_chunk(0)
        for k in range(n_items):
            wait_chunk(pend)
            out_copy(k).start()
            if k + 1 < n_items:
                if k >= 1:
                    out_copy(k - 1).wait()
                pend = gather_chunk(k + 1)
        if n_items >= 2:
            out_copy(n_items - 2).wait()
        out_copy(n_items - 1).wait()

    return copy_k(x)


# speedup vs baseline: 4.2410x; 1.0013x over previous
import functools

import jax
import jax.numpy as jnp
from jax import lax
from jax.experimental import pallas as pl
from jax.experimental.pallas import tpu as pltpu
from jax.experimental.pallas import tpu_sc as plsc


def kernel(x):
    B, S, C = x.shape
    h = int(round(S ** 0.5))
    hi, hj = h - 1, h - 2
    K = hi * hj
    seg = hj * C

    xf = x.reshape(B * S * C)

    info = plsc.get_sparse_core_info()
    nw = info.num_cores * info.num_subcores
    per_w = B // nw

    GR = 8
    chunks = []
    i0 = 0
    while i0 < hi:
        chunks.append((i0, min(GR, hi - i0)))
        i0 += GR
    items = [(t, c0, ngr) for t in range(per_w) for (c0, ngr) in chunks]
    n_items = len(items)

    mesh = plsc.VectorSubcoreMesh(core_axis_name="c", subcore_axis_name="s")

    @functools.partial(
        pl.kernel,
        mesh=mesh,
        out_type=jax.ShapeDtypeStruct((B * K * C,), jnp.float32),
        scratch_types=[
            pltpu.VMEM((GR * seg,), jnp.float32),
            pltpu.VMEM((GR * seg,), jnp.float32),
            pltpu.SemaphoreType.DMA,
            pltpu.SemaphoreType.DMA,
            pltpu.SemaphoreType.DMA,
            pltpu.SemaphoreType.DMA,
        ],
        compiler_params=pltpu.CompilerParams(use_tc_tiling_on_sc=False),
    )
    def copy_k(x_hbm, out_hbm, ob0, ob1, is0, is1, os0, os1):
        wid = lax.axis_index("s") * info.num_cores + lax.axis_index("c")
        obufs = (ob0, ob1)
        isems, osems = (is0, is1), (os0, os1)

        def gather_chunk(k):
            t, c0, ngr = items[k]
            b = wid * per_w + t
            cps = [
                pltpu.make_async_copy(
                    x_hbm.at[pl.ds(b * S * C + ((c0 + q) * h + 1) * C, seg)],
                    obufs[k % 2].at[pl.ds(q * seg, seg)],
                    isems[k % 2],
                )
                for q in range(ngr)
            ]
            for cp in cps:
                cp.start()
            return cps

        def wait_chunk(cps):
            for cp in cps:
                cp.wait()

        def out_copy(k):
            t, c0, ngr = items[k]
            b = wid * per_w + t
            return pltpu.make_async_copy(
                obufs[k % 2].at[pl.ds(0, ngr * seg)],
                out_hbm.at[pl.ds(b * K * C + c0 * seg, ngr * seg)],
                osems[k % 2],
            )

        pend = gather_chunk(0)
        for k in range(n_items):
            wait_chunk(pend)
            out_copy(k).start()
            if k + 1 < n_items:
                if k >= 1:
                    out_copy(k - 1).wait()
                pend = gather_chunk(k + 1)
        if n_items >= 2:
            out_copy(n_items - 2).wait()
        out_copy(n_items - 1).wait()

    return copy_k(xf).reshape(B, K, C)


# speedup vs baseline: 8.4605x; 1.9949x over previous
import functools

import jax
import jax.numpy as jnp
from jax import lax
from jax.experimental import pallas as pl
from jax.experimental.pallas import tpu as pltpu
from jax.experimental.pallas import tpu_sc as plsc


def kernel(x):
    B, S, C = x.shape
    h = int(round(S ** 0.5))
    hi, hj = h - 1, h - 2
    K = hi * hj

    info = plsc.get_sparse_core_info()
    nw = info.num_cores * info.num_subcores
    per_w = B // nw

    GR = 4
    chunks = []
    i0 = 0
    while i0 < hi:
        chunks.append((i0, min(GR, hi - i0)))
        i0 += GR
    items = [(t, c0, ngr) for t in range(per_w) for (c0, ngr) in chunks]
    n_items = len(items)
    NC16 = C // 16
    TAILN = (hi % GR) * hj
    TAILA = TAILN - (TAILN % 8)

    mesh = plsc.VectorSubcoreMesh(core_axis_name="c", subcore_axis_name="s")

    @functools.partial(
        pl.kernel,
        mesh=mesh,
        out_type=jax.ShapeDtypeStruct((B, K, C), jnp.float32),
        scratch_types=[
            pltpu.VMEM((GR * h, C), jnp.float32),
            pltpu.VMEM((GR * h, C), jnp.float32),
            pltpu.VMEM((GR * hj, C), jnp.float32),
            pltpu.VMEM((GR * hj, C), jnp.float32),
            pltpu.VMEM((TAILN - TAILA, C), jnp.float32),
            pltpu.SemaphoreType.DMA,
            pltpu.SemaphoreType.DMA,
            pltpu.SemaphoreType.DMA,
            pltpu.SemaphoreType.DMA,
            pltpu.SemaphoreType.DMA,
        ],
        compiler_params=pltpu.CompilerParams(needs_layout_passes=False),
    )
    def copy_k(x_hbm, out_hbm, ib0, ib1, ob0, ob1, tb,
               is0, is1, os0, os1, tsem):
        wid = lax.axis_index("s") * info.num_cores + lax.axis_index("c")
        ibufs, obufs = (ib0, ib1), (ob0, ob1)
        isems, osems = (is0, is1), (os0, os1)

        def is_tail(k):
            return items[k][2] != GR

        def in_copy(k):
            t, c0, ngr = items[k]
            b = wid * per_w + t
            return pltpu.make_async_copy(
                x_hbm.at[b, pl.ds(c0 * h, ngr * h), :],
                ibufs[k % 2].at[pl.ds(0, ngr * h)],
                isems[k % 2],
            )

        def ob_copy(k):
            t, c0, ngr = items[k]
            b = wid * per_w + t
            n = GR * hj if ngr == GR else TAILA
            return pltpu.make_async_copy(
                obufs[k % 2].at[pl.ds(0, n)],
                out_hbm.at[b, pl.ds(c0 * hj, n), :],
                osems[k % 2],
            )

        def tb_copy(k):
            t, c0, ngr = items[k]
            b = wid * per_w + t
            return pltpu.make_async_copy(
                tb,
                out_hbm.at[b, pl.ds(c0 * hj + TAILA, TAILN - TAILA), :],
                tsem,
            )

        def repack(k):
            _, _, ngr = items[k]
            ib, ob = ibufs[k % 2], obufs[k % 2]
            iota16 = lax.iota(jnp.int32, 16)

            def copy_row(r, dst_ref, dst_row):
                src = (r // hj) * h + (r % hj) + 1
                src_rows = jnp.full((16,), src, jnp.int32)
                dst_rows = jnp.full((16,), dst_row, jnp.int32)
                for c in range(NC16):
                    cols = iota16 + (c * 16)
                    v = plsc.load_gather(ib, [src_rows, cols])
                    plsc.store_scatter(dst_ref, [dst_rows, cols], v)

            def body(r, _):
                copy_row(r, ob, r)
                return 0

            n_main = ngr * hj if ngr == GR else TAILA
            lax.fori_loop(0, n_main, body, 0)
            if ngr != GR:
                for r in range(TAILA, TAILN):
                    copy_row(jnp.int32(r), tb, jnp.int32(r - TAILA))

        prev_out = {}
        last_tail = None
        in_copy(0).start()
        for k in range(n_items):
            slot = k % 2
            in_copy(k).wait()
            if k + 1 < n_items:
                in_copy(k + 1).start()
            if slot in prev_out:
                ob_copy(prev_out[slot]).wait()
            if is_tail(k) and last_tail is not None:
                tb_copy(last_tail).wait()
            repack(k)
            ob_copy(k).start()
            prev_out[slot] = k
            if is_tail(k):
                tb_copy(k).start()
                last_tail = k
        for slot in prev_out:
            ob_copy(prev_out[slot]).wait()
        if last_tail is not None:
            tb_copy(last_tail).wait()

    return copy_k(x)
